# baseline (speedup 1.0000x reference)
import jax
import jax.numpy as jnp
from jax.experimental import pallas as pl

_BLOCK_T = 512


def _router_block(x_ref, wt_ref, b_ref, out_ref):
    logits = jax.lax.dot_general(
        x_ref[...], wt_ref[...],
        dimension_numbers=(((1,), (0,)), ((), ())),
        preferred_element_type=jnp.float32,
    )
    logits = logits + b_ref[...]
    m = jnp.max(logits, axis=-1, keepdims=True)
    e = jnp.exp(logits - m)
    out_ref[...] = e / jnp.sum(e, axis=-1, keepdims=True)


def kernel(x, W, b):
    n_tokens, hidden = x.shape
    n_experts = W.shape[0]
    block_t = min(_BLOCK_T, n_tokens)
    wt = W.T
    return pl.pallas_call(
        _router_block,
        grid=(n_tokens // block_t,),
        in_specs=[
            pl.BlockSpec((block_t, hidden), lambda i: (i, 0)),
            pl.BlockSpec((hidden, n_experts), lambda i: (0, 0)),
            pl.BlockSpec((1, n_experts), lambda i: (0, 0)),
        ],
        out_specs=pl.BlockSpec((block_t, n_experts), lambda i: (i, 0)),
        out_shape=jax.ShapeDtypeStruct((n_tokens, n_experts), jnp.float32),
    )(x, wt, b.reshape(1, n_experts))


# speedup vs baseline: 1.2349x; 1.2349x over previous
import jax
import jax.numpy as jnp
from jax.experimental import pallas as pl

_BLOCK_T = 1024


def _router_block(x_ref, w_ref, b_ref, out_ref):
    logits = jax.lax.dot_general(
        x_ref[...], w_ref[...],
        dimension_numbers=(((1,), (1,)), ((), ())),
        preferred_element_type=jnp.float32,
    )
    logits = logits + b_ref[...]
    m = jnp.max(logits, axis=-1, keepdims=True)
    e = jnp.exp(logits - m)
    out_ref[...] = e / jnp.sum(e, axis=-1, keepdims=True)


def kernel(x, W, b):
    n_tokens, hidden = x.shape
    n_experts = W.shape[0]
    block_t = min(_BLOCK_T, n_tokens)
    return pl.pallas_call(
        _router_block,
        grid=(n_tokens // block_t,),
        in_specs=[
            pl.BlockSpec((block_t, hidden), lambda i: (i, 0)),
            pl.BlockSpec((n_experts, hidden), lambda i: (0, 0)),
            pl.BlockSpec((1, n_experts), lambda i: (0, 0)),
        ],
        out_specs=pl.BlockSpec((block_t, n_experts), lambda i: (i, 0)),
        out_shape=jax.ShapeDtypeStruct((n_tokens, n_experts), jnp.float32),
    )(x, W, b.reshape(1, n_experts))
